# Initial kernel scaffold; baseline (speedup 1.0000x reference)
import jax
import jax.numpy as jnp
from jax.experimental import pallas as pl


def kernel(x, point_pairs, cluster_ids, recons, pts):
    raise NotImplementedError("write your pallas kernel here")


# speedup vs baseline: 22.9531x; 22.9531x over previous
import functools

import jax
import jax.numpy as jnp
from jax import lax
from jax.experimental import pallas as pl
from jax.experimental.pallas import tpu as pltpu
from jax.experimental.pallas import tpu_sc as plsc

_TAU = 0.1
_SIGMA = 0.75
_PT = 0.9
_SHIFT = 10.0

_NC = 2
_NS = 16
_L = 16
_NW = _NC * _NS
_CH = 2000


def _dist_block_kernel(a_ref, b_ref, o_ref):
    a = a_ref[...]
    b = b_ref[...]
    ab = lax.dot_general(a, b, (((1,), (1,)), ((), ())),
                         preferred_element_type=jnp.float32)
    sqa = jnp.sum(a * a, axis=1, keepdims=True)
    sqb = jnp.sum(b * b, axis=1)[None, :]
    o_ref[...] = sqa + sqb - 2.0 * ab


def _sc_pass1_body(q_hbm, p0_hbm, p1_hbm, cid_hbm, rec_hbm, pts_hbm,
                   es_hbm, pos_hbm, denp_hbm,
                   cid_v, rec_v, pts_v, p0_v, p1_v, qi_v, q_v, es_v, pos_v,
                   nes_v, den_sh, sem, *, n_points, n_pairs):
    cidx = lax.axis_index("c")
    sidx = lax.axis_index("s")
    wid = sidx * _NC + cidx
    epw = n_pairs // _NW
    base = wid * epw
    nv = _CH // _L

    pltpu.sync_copy(cid_hbm, cid_v)
    pltpu.sync_copy(rec_hbm, rec_v)
    pltpu.sync_copy(pts_hbm, pts_v)

    @pl.when(sidx == 0)
    def _():
        def zb(i, c):
            es_v[pl.ds(i * _L, _L)] = jnp.zeros((_L,), jnp.float32)
            return c

        lax.fori_loop(0, nv, zb, 0)

        def zs(i, c):
            pltpu.sync_copy(es_v, den_sh.at[pl.ds(i * _CH, _CH)])
            return c

        lax.fori_loop(0, n_points // _CH, zs, 0)

    plsc.subcore_barrier()

    def chunk_body(k, c):
        off = base + k * _CH
        pltpu.sync_copy(p0_hbm.at[pl.ds(off, _CH)], p0_v)
        pltpu.sync_copy(p1_hbm.at[pl.ds(off, _CH)], p1_v)

        def idx_body(i, cc):
            sl = pl.ds(i * _L, _L)
            qi_v[sl] = p0_v[sl] * n_points + p1_v[sl]
            return cc

        lax.fori_loop(0, nv, idx_body, 0)

        pltpu.async_copy(q_hbm.at[qi_v], q_v, sem).wait()

        def cmp_body(i, cc):
            sl = pl.ds(i * _L, _L)
            p0c = p0_v[sl]
            p1c = p1_v[sl]
            c0 = plsc.load_gather(cid_v, [p0c])
            c1 = plsc.load_gather(cid_v, [p1c])
            r0 = plsc.load_gather(rec_v, [p0c])
            r1 = plsc.load_gather(rec_v, [p1c])
            t0 = plsc.load_gather(pts_v, [p0c])
            t1 = plsc.load_gather(pts_v, [p1c])
            pos = (c0 == c1) & (r0 > 0) & (r1 > 0) & (t0 > _PT) & (t1 > _PT)
            posf = jnp.where(pos, 1.0, 0.0)
            q = jnp.maximum(q_v[sl], 1e-20)
            ib = plsc.bitcast(q, jnp.int32)
            t = plsc.bitcast(0x5F3759DF - (ib >> 1), jnp.float32)
            t = t * (1.5 - 0.5 * q * t * t)
            t = t * (1.5 - 0.5 * q * t * t)
            t = t * (1.5 - 0.5 * q * t * t)
            d = q * t
            s = (1.0 / _TAU) * jnp.exp(d * (-1.0 / (2.0 * _SIGMA * _SIGMA)))
            es = jnp.exp(s - _SHIFT)
            es_v[sl] = es
            pos_v[sl] = posf
            nes_v[sl] = es * (1.0 - posf)
            return cc

        lax.fori_loop(0, nv, cmp_body, 0)

        pltpu.sync_copy(nes_v, den_sh.at[p0_v], add=True)
        pltpu.sync_copy(es_v, es_hbm.at[pl.ds(off, _CH)])
        pltpu.sync_copy(pos_v, pos_hbm.at[pl.ds(off, _CH)])
        return c

    lax.fori_loop(0, epw // _CH, chunk_body, 0)

    plsc.subcore_barrier()

    @pl.when(sidx == 0)
    def _():
        pltpu.sync_copy(den_sh, denp_hbm.at[cidx])


def _ln_poly(r):
    ib = plsc.bitcast(r, jnp.int32)
    e = ((ib >> 23) & 0xFF) - 127
    m = plsc.bitcast((ib & 0x7FFFFF) | (127 << 23), jnp.float32)
    big = m > 1.4142135623730951
    m = jnp.where(big, m * 0.5, m)
    ef = (e + jnp.where(big, 1, 0)).astype(jnp.float32)
    z = (m - 1.0) / (m + 1.0)
    z2 = z * z
    p = z * (2.0 + z2 * (0.66666667 + z2 * (0.4 + z2 * (0.28571429 + z2 * 0.22222222))))
    return ef * 0.6931471805599453 + p


def _sc_pass2_body(es_hbm, pos_hbm, p0_hbm, cid_hbm, denp_hbm,
                   labp_hbm,
                   den_v, dpb_v, cden_v, cid_v, p0_v, es_v, pos_v,
                   acc_v, cnt_v, row_v, *, n_points, n_pairs):
    cidx = lax.axis_index("c")
    sidx = lax.axis_index("s")
    wid = sidx * _NC + cidx
    epw = n_pairs // _NW
    base = wid * epw
    nv = _CH // _L
    nv_n = n_points // _L

    pltpu.sync_copy(cid_hbm, cid_v)
    pltpu.sync_copy(denp_hbm.at[0], den_v)
    pltpu.sync_copy(denp_hbm.at[1], dpb_v)

    def merge_body(i, c):
        sl = pl.ds(i * _L, _L)
        den_v[sl] = den_v[sl] + dpb_v[sl]
        return c

    lax.fori_loop(0, nv_n, merge_body, 0)

    def rank_body(i, carry):
        sl = pl.ds(i * _L, _L)
        dchunk = den_v[sl]
        pres = dchunk > 0.0
        presi = jnp.where(pres, 1, 0)
        cum = plsc.cumsum(presi)
        rank = carry + cum - 1
        plsc.store_scatter(cden_v, [rank], dchunk, mask=pres)
        return carry + jnp.max(cum)

    ng = lax.fori_loop(0, nv_n, rank_body, jnp.int32(0))
    ngm = jnp.maximum(ng, 1)

    def zb(i, c):
        sl = pl.ds(i * _L, _L)
        acc_v[sl] = jnp.zeros((_L,), jnp.float32)
        cnt_v[sl] = jnp.zeros((_L,), jnp.float32)
        return c

    lax.fori_loop(0, 1024 // _L, zb, 0)

    lane = lax.broadcasted_iota(jnp.int32, (_L,), 0)

    def chunk_body(k, c):
        off = base + k * _CH
        pltpu.sync_copy(p0_hbm.at[pl.ds(off, _CH)], p0_v)
        pltpu.sync_copy(es_hbm.at[pl.ds(off, _CH)], es_v)
        pltpu.sync_copy(pos_hbm.at[pl.ds(off, _CH)], pos_v)

        def body(i, cc):
            sl = pl.ds(i * _L, _L)
            p0c = p0_v[sl]
            num = es_v[sl]
            posf = pos_v[sl]
            di = jnp.minimum(p0c, ngm - 1)
            den = plsc.load_gather(cden_v, [di])
            r = num / (num + den)
            loss = -_ln_poly(r) * posf
            lab = jnp.clip(plsc.load_gather(cid_v, [p0c]), 0, 63)
            slot = lab * _L + lane
            plsc.addupdate_scatter(acc_v, [slot], loss)
            plsc.addupdate_scatter(cnt_v, [slot], posf)
            return cc

        lax.fori_loop(0, nv, body, 0)
        return c

    lax.fori_loop(0, epw // _CH, chunk_body, 0)

    def fold_body(g, c):
        base_lab = g * _L
        idx0 = (base_lab + lane) * _L

        def inner(k, vs):
            va, vc = vs
            va = va + plsc.load_gather(acc_v, [idx0 + k])
            vc = vc + plsc.load_gather(cnt_v, [idx0 + k])
            return (va, vc)

        va, vc = lax.fori_loop(
            0, _L, inner,
            (jnp.zeros((_L,), jnp.float32), jnp.zeros((_L,), jnp.float32)))
        row_v[pl.ds(base_lab, _L)] = va
        row_v[pl.ds(64 + base_lab, _L)] = vc
        return c

    lax.fori_loop(0, 64 // _L, fold_body, 0)

    pltpu.sync_copy(row_v, labp_hbm.at[wid])


def _fold_kernel(a_ref, o_ref):
    a = a_ref[...]
    srow = jnp.sum(a, axis=0, keepdims=True)
    ls = srow[:, :64]
    cs = srow[:, 64:]
    pres = cs > 0.0
    mean = jnp.where(pres, ls / jnp.where(pres, cs, 1.0), 0.0)
    total = jnp.sum(mean)
    nl = jnp.sum(jnp.where(pres, 1.0, 0.0))
    o_ref[0, 0] = total / nl


@jax.jit
def _impl(x, point_pairs, cluster_ids, recons, pts):
    n, d = x.shape
    e = point_pairs.shape[1]
    bm = bn = 512

    q = pl.pallas_call(
        _dist_block_kernel,
        grid=(pl.cdiv(n, bm), pl.cdiv(n, bn)),
        in_specs=[pl.BlockSpec((bm, d), lambda i, j: (i, 0)),
                  pl.BlockSpec((bn, d), lambda i, j: (j, 0))],
        out_specs=pl.BlockSpec((bm, bn), lambda i, j: (i, j)),
        out_shape=jax.ShapeDtypeStruct((n, n), jnp.float32),
    )(x, x)
    qflat = q.reshape(n * n)

    p0 = point_pairs[0]
    p1 = point_pairs[1]
    cid = cluster_ids.astype(jnp.int32)
    rec = recons.astype(jnp.int32)
    pts_f = pts.astype(jnp.float32)

    mesh = plsc.VectorSubcoreMesh(core_axis_name="c", subcore_axis_name="s",
                                  num_cores=_NC, num_subcores=_NS)
    sc_params = pltpu.CompilerParams(needs_layout_passes=False)

    es, pos, denp = pl.kernel(
        functools.partial(_sc_pass1_body, n_points=n, n_pairs=e),
        out_type=[jax.ShapeDtypeStruct((e,), jnp.float32),
                  jax.ShapeDtypeStruct((e,), jnp.float32),
                  jax.ShapeDtypeStruct((_NC, n), jnp.float32)],
        mesh=mesh,
        scratch_types=[
            pltpu.VMEM((n,), jnp.int32),
            pltpu.VMEM((n,), jnp.int32),
            pltpu.VMEM((n,), jnp.float32),
            pltpu.VMEM((_CH,), jnp.int32),
            pltpu.VMEM((_CH,), jnp.int32),
            pltpu.VMEM((_CH,), jnp.int32),
            pltpu.VMEM((_CH,), jnp.float32),
            pltpu.VMEM((_CH,), jnp.float32),
            pltpu.VMEM((_CH,), jnp.float32),
            pltpu.VMEM((_CH,), jnp.float32),
            pltpu.VMEM_SHARED((n,), jnp.float32),
            pltpu.SemaphoreType.DMA,
        ],
        compiler_params=sc_params,
    )(qflat, p0, p1, cid, rec, pts_f)

    labp = pl.kernel(
        functools.partial(_sc_pass2_body, n_points=n, n_pairs=e),
        out_type=[jax.ShapeDtypeStruct((_NW, 128), jnp.float32)],
        mesh=mesh,
        scratch_types=[
            pltpu.VMEM((n,), jnp.float32),
            pltpu.VMEM((n,), jnp.float32),
            pltpu.VMEM((n,), jnp.float32),
            pltpu.VMEM((n,), jnp.int32),
            pltpu.VMEM((_CH,), jnp.int32),
            pltpu.VMEM((_CH,), jnp.float32),
            pltpu.VMEM((_CH,), jnp.float32),
            pltpu.VMEM((1024,), jnp.float32),
            pltpu.VMEM((1024,), jnp.float32),
            pltpu.VMEM((128,), jnp.float32),
        ],
        compiler_params=sc_params,
    )(es, pos, p0, cid, denp)[0]

    out = pl.pallas_call(
        _fold_kernel,
        in_specs=[pl.BlockSpec((_NW, 128), lambda: (0, 0))],
        out_specs=pl.BlockSpec(memory_space=pltpu.SMEM),
        out_shape=jax.ShapeDtypeStruct((1, 1), jnp.float32),
    )(labp)
    return out[0, 0]


def kernel(x, point_pairs, cluster_ids, recons, pts):
    return _impl(x, point_pairs, cluster_ids, recons, pts)


# speedup vs baseline: 39.8821x; 1.7375x over previous
import functools

import jax
import jax.numpy as jnp
from jax import lax
from jax.experimental import pallas as pl
from jax.experimental.pallas import tpu as pltpu
from jax.experimental.pallas import tpu_sc as plsc

_TAU = 0.1
_SIGMA = 0.75
_PT = 0.9
_SHIFT = 10.0

_NC = 2
_NS = 16
_L = 16
_NW = _NC * _NS
_CH = 2000


def _dist_block_kernel(a_ref, b_ref, o_ref):
    a = a_ref[...]
    b = b_ref[...]
    ab = lax.dot_general(a, b, (((1,), (1,)), ((), ())),
                         preferred_element_type=jnp.float32)
    sqa = jnp.sum(a * a, axis=1, keepdims=True)
    sqb = jnp.sum(b * b, axis=1)[None, :]
    q = sqa + sqb - 2.0 * ab
    for t in range(4):
        o_ref[0, pl.ds(t * 512, 512), :] = q[:, t * 128:(t + 1) * 128]


def _sc_pass1_body(q_hbm, p0_hbm, p1_hbm, cid_hbm, rec_hbm, pts_hbm,
                   es_hbm, pos_hbm, denp_hbm,
                   cid_v, rec_v, pts_v, p0_v, p1_v, qi_v, q_v, es_v, pos_v,
                   nes_v, den_sh, sem, *, n_points, n_pairs, n_blocks):
    cidx = lax.axis_index("c")
    sidx = lax.axis_index("s")
    wid = sidx * _NC + cidx
    epw = n_pairs // _NW
    base = wid * epw
    nv = _CH // _L

    pltpu.sync_copy(cid_hbm, cid_v)
    pltpu.sync_copy(rec_hbm, rec_v)
    pltpu.sync_copy(pts_hbm, pts_v)

    @pl.when(sidx == 0)
    def _():
        def zb(i, c):
            es_v[pl.ds(i * _L, _L)] = jnp.zeros((_L,), jnp.float32)
            return c

        lax.fori_loop(0, nv, zb, 0)

        def zs(i, c):
            pltpu.sync_copy(es_v, den_sh.at[pl.ds(i * _CH, _CH)])
            return c

        lax.fori_loop(0, n_points // _CH, zs, 0)

    plsc.subcore_barrier()

    def chunk_body(k, c):
        off = base + k * _CH
        pltpu.sync_copy(p0_hbm.at[pl.ds(off, _CH)], p0_v)
        pltpu.sync_copy(p1_hbm.at[pl.ds(off, _CH)], p1_v)

        def idx_body(i, cc):
            sl = pl.ds(i * _L, _L)
            p0c = p0_v[sl]
            p1c = p1_v[sl]
            bi = p0c >> 9
            bj = p1c >> 9
            rr = p0c & 511
            cc_ = p1c & 511
            qi_v[sl] = ((bi * n_blocks + bj) * 262144
                        + ((cc_ >> 7) * 512 + rr) * 128 + (cc_ & 127))
            return cc

        lax.fori_loop(0, nv, idx_body, 0)

        pltpu.async_copy(q_hbm.at[qi_v], q_v, sem).wait()

        def cmp_body(i, cc):
            sl = pl.ds(i * _L, _L)
            p0c = p0_v[sl]
            p1c = p1_v[sl]
            c0 = plsc.load_gather(cid_v, [p0c])
            c1 = plsc.load_gather(cid_v, [p1c])
            r0 = plsc.load_gather(rec_v, [p0c])
            r1 = plsc.load_gather(rec_v, [p1c])
            t0 = plsc.load_gather(pts_v, [p0c])
            t1 = plsc.load_gather(pts_v, [p1c])
            pos = (c0 == c1) & (r0 > 0) & (r1 > 0) & (t0 > _PT) & (t1 > _PT)
            posf = jnp.where(pos, 1.0, 0.0)
            q = jnp.maximum(q_v[sl], 1e-20)
            ib = plsc.bitcast(q, jnp.int32)
            t = plsc.bitcast(0x5F3759DF - (ib >> 1), jnp.float32)
            t = t * (1.5 - 0.5 * q * t * t)
            t = t * (1.5 - 0.5 * q * t * t)
            t = t * (1.5 - 0.5 * q * t * t)
            d = q * t
            s = (1.0 / _TAU) * jnp.exp(d * (-1.0 / (2.0 * _SIGMA * _SIGMA)))
            es = jnp.exp(s - _SHIFT)
            es_v[sl] = es
            pos_v[sl] = posf
            nes_v[sl] = es * (1.0 - posf)
            return cc

        lax.fori_loop(0, nv, cmp_body, 0)

        pltpu.sync_copy(nes_v, den_sh.at[p0_v], add=True)
        pltpu.sync_copy(es_v, es_hbm.at[pl.ds(off, _CH)])
        pltpu.sync_copy(pos_v, pos_hbm.at[pl.ds(off, _CH)])
        return c

    lax.fori_loop(0, epw // _CH, chunk_body, 0)

    plsc.subcore_barrier()

    @pl.when(sidx == 0)
    def _():
        pltpu.sync_copy(den_sh, denp_hbm.at[cidx])


def _ln_poly(r):
    ib = plsc.bitcast(r, jnp.int32)
    e = ((ib >> 23) & 0xFF) - 127
    m = plsc.bitcast((ib & 0x7FFFFF) | (127 << 23), jnp.float32)
    big = m > 1.4142135623730951
    m = jnp.where(big, m * 0.5, m)
    ef = (e + jnp.where(big, 1, 0)).astype(jnp.float32)
    z = (m - 1.0) / (m + 1.0)
    z2 = z * z
    p = z * (2.0 + z2 * (0.66666667 + z2 * (0.4 + z2 * (0.28571429 + z2 * 0.22222222))))
    return ef * 0.6931471805599453 + p


def _sc_pass2_body(es_hbm, pos_hbm, p0_hbm, cid_hbm, denp_hbm,
                   labp_hbm,
                   den_v, dpb_v, cden_v, cid_v, p0_v, es_v, pos_v,
                   acc_v, cnt_v, row_v, *, n_points, n_pairs):
    cidx = lax.axis_index("c")
    sidx = lax.axis_index("s")
    wid = sidx * _NC + cidx
    epw = n_pairs // _NW
    base = wid * epw
    nv = _CH // _L
    nv_n = n_points // _L

    pltpu.sync_copy(cid_hbm, cid_v)
    pltpu.sync_copy(denp_hbm.at[0], den_v)
    pltpu.sync_copy(denp_hbm.at[1], dpb_v)

    def merge_body(i, c):
        sl = pl.ds(i * _L, _L)
        den_v[sl] = den_v[sl] + dpb_v[sl]
        return c

    lax.fori_loop(0, nv_n, merge_body, 0)

    def rank_body(i, carry):
        sl = pl.ds(i * _L, _L)
        dchunk = den_v[sl]
        pres = dchunk > 0.0
        presi = jnp.where(pres, 1, 0)
        cum = plsc.cumsum(presi)
        rank = carry + cum - 1
        plsc.store_scatter(cden_v, [rank], dchunk, mask=pres)
        return carry + jnp.max(cum)

    ng = lax.fori_loop(0, nv_n, rank_body, jnp.int32(0))
    ngm = jnp.maximum(ng, 1)

    def zb(i, c):
        sl = pl.ds(i * _L, _L)
        acc_v[sl] = jnp.zeros((_L,), jnp.float32)
        cnt_v[sl] = jnp.zeros((_L,), jnp.float32)
        return c

    lax.fori_loop(0, 1024 // _L, zb, 0)

    lane = lax.broadcasted_iota(jnp.int32, (_L,), 0)

    def chunk_body(k, c):
        off = base + k * _CH
        pltpu.sync_copy(p0_hbm.at[pl.ds(off, _CH)], p0_v)
        pltpu.sync_copy(es_hbm.at[pl.ds(off, _CH)], es_v)
        pltpu.sync_copy(pos_hbm.at[pl.ds(off, _CH)], pos_v)

        def body(i, cc):
            sl = pl.ds(i * _L, _L)
            p0c = p0_v[sl]
            num = es_v[sl]
            posf = pos_v[sl]
            di = jnp.minimum(p0c, ngm - 1)
            den = plsc.load_gather(cden_v, [di])
            r = num / (num + den)
            loss = -_ln_poly(r) * posf
            lab = jnp.clip(plsc.load_gather(cid_v, [p0c]), 0, 63)
            slot = lab * _L + lane
            plsc.addupdate_scatter(acc_v, [slot], loss)
            plsc.addupdate_scatter(cnt_v, [slot], posf)
            return cc

        lax.fori_loop(0, nv, body, 0)
        return c

    lax.fori_loop(0, epw // _CH, chunk_body, 0)

    def fold_body(g, c):
        base_lab = g * _L
        idx0 = (base_lab + lane) * _L

        def inner(k, vs):
            va, vc = vs
            va = va + plsc.load_gather(acc_v, [idx0 + k])
            vc = vc + plsc.load_gather(cnt_v, [idx0 + k])
            return (va, vc)

        va, vc = lax.fori_loop(
            0, _L, inner,
            (jnp.zeros((_L,), jnp.float32), jnp.zeros((_L,), jnp.float32)))
        row_v[pl.ds(base_lab, _L)] = va
        row_v[pl.ds(64 + base_lab, _L)] = vc
        return c

    lax.fori_loop(0, 64 // _L, fold_body, 0)

    pltpu.sync_copy(row_v, labp_hbm.at[wid])


def _fold_kernel(a_ref, o_ref):
    a = a_ref[...]
    srow = jnp.sum(a, axis=0, keepdims=True)
    ls = srow[:, :64]
    cs = srow[:, 64:]
    pres = cs > 0.0
    mean = jnp.where(pres, ls / jnp.where(pres, cs, 1.0), 0.0)
    total = jnp.sum(mean)
    nl = jnp.sum(jnp.where(pres, 1.0, 0.0))
    o_ref[0, 0] = total / nl


@jax.jit
def _impl(x, point_pairs, cluster_ids, recons, pts):
    n, d = x.shape
    e = point_pairs.shape[1]
    bm = bn = 512

    nb = pl.cdiv(n, bm)
    q3 = pl.pallas_call(
        _dist_block_kernel,
        grid=(nb, nb),
        in_specs=[pl.BlockSpec((bm, d), lambda i, j: (i, 0)),
                  pl.BlockSpec((bn, d), lambda i, j: (j, 0))],
        out_specs=pl.BlockSpec((1, 4 * bm, 128),
                               lambda i, j, _nb=nb: (i * _nb + j, 0, 0)),
        out_shape=jax.ShapeDtypeStruct((nb * nb, 4 * bm, 128), jnp.float32),
    )(x, x)
    qflat = q3.reshape(nb * nb * 4 * bm * 128)

    p0 = point_pairs[0]
    p1 = point_pairs[1]
    cid = cluster_ids.astype(jnp.int32)
    rec = recons.astype(jnp.int32)
    pts_f = pts.astype(jnp.float32)

    mesh = plsc.VectorSubcoreMesh(core_axis_name="c", subcore_axis_name="s",
                                  num_cores=_NC, num_subcores=_NS)
    sc_params = pltpu.CompilerParams(needs_layout_passes=False)

    es, pos, denp = pl.kernel(
        functools.partial(_sc_pass1_body, n_points=n, n_pairs=e, n_blocks=nb),
        out_type=[jax.ShapeDtypeStruct((e,), jnp.float32),
                  jax.ShapeDtypeStruct((e,), jnp.float32),
                  jax.ShapeDtypeStruct((_NC, n), jnp.float32)],
        mesh=mesh,
        scratch_types=[
            pltpu.VMEM((n,), jnp.int32),
            pltpu.VMEM((n,), jnp.int32),
            pltpu.VMEM((n,), jnp.float32),
            pltpu.VMEM((_CH,), jnp.int32),
            pltpu.VMEM((_CH,), jnp.int32),
            pltpu.VMEM((_CH,), jnp.int32),
            pltpu.VMEM((_CH,), jnp.float32),
            pltpu.VMEM((_CH,), jnp.float32),
            pltpu.VMEM((_CH,), jnp.float32),
            pltpu.VMEM((_CH,), jnp.float32),
            pltpu.VMEM_SHARED((n,), jnp.float32),
            pltpu.SemaphoreType.DMA,
        ],
        compiler_params=sc_params,
    )(qflat, p0, p1, cid, rec, pts_f)

    labp = pl.kernel(
        functools.partial(_sc_pass2_body, n_points=n, n_pairs=e),
        out_type=[jax.ShapeDtypeStruct((_NW, 128), jnp.float32)],
        mesh=mesh,
        scratch_types=[
            pltpu.VMEM((n,), jnp.float32),
            pltpu.VMEM((n,), jnp.float32),
            pltpu.VMEM((n,), jnp.float32),
            pltpu.VMEM((n,), jnp.int32),
            pltpu.VMEM((_CH,), jnp.int32),
            pltpu.VMEM((_CH,), jnp.float32),
            pltpu.VMEM((_CH,), jnp.float32),
            pltpu.VMEM((1024,), jnp.float32),
            pltpu.VMEM((1024,), jnp.float32),
            pltpu.VMEM((128,), jnp.float32),
        ],
        compiler_params=sc_params,
    )(es, pos, p0, cid, denp)[0]

    out = pl.pallas_call(
        _fold_kernel,
        in_specs=[pl.BlockSpec((_NW, 128), lambda: (0, 0))],
        out_specs=pl.BlockSpec(memory_space=pltpu.SMEM),
        out_shape=jax.ShapeDtypeStruct((1, 1), jnp.float32),
    )(labp)
    return out[0, 0]


def kernel(x, point_pairs, cluster_ids, recons, pts):
    return _impl(x, point_pairs, cluster_ids, recons, pts)


# speedup vs baseline: 59.6325x; 1.4952x over previous
import functools

import jax
import jax.numpy as jnp
from jax import lax
from jax.experimental import pallas as pl
from jax.experimental.pallas import tpu as pltpu
from jax.experimental.pallas import tpu_sc as plsc

_TAU = 0.1
_SIGMA = 0.75
_PT = 0.9
_SHIFT = 10.0

_NC = 2
_NS = 16
_L = 16
_NW = _NC * _NS
_CH = 2000


def _dist_block_kernel(bi_ref, bj_ref, a_ref, b_ref, o_ref):
    a = a_ref[...]
    b = b_ref[...]
    ab = lax.dot_general(a, b, (((1,), (1,)), ((), ())),
                         preferred_element_type=jnp.float32)
    sqa = jnp.sum(a * a, axis=1, keepdims=True)
    sqb = jnp.sum(b * b, axis=1)[None, :]
    q = sqa + sqb - 2.0 * ab
    for t in range(4):
        o_ref[0, pl.ds(t * 512, 512), :] = q[:, t * 128:(t + 1) * 128]


def _sc_pass1_body(q_hbm, p0_hbm, p1_hbm, cid_hbm, rec_hbm, pts_hbm,
                   es_hbm, pos_hbm, denp_hbm,
                   cid_v, rec_v, pts_v, p0_v, p1_v, qi_v, q_v, es_v, pos_v,
                   nes_v, den_sh, sem, *, n_points, n_pairs, n_blocks):
    cidx = lax.axis_index("c")
    sidx = lax.axis_index("s")
    wid = sidx * _NC + cidx
    epw = n_pairs // _NW
    base = wid * epw
    nv = _CH // _L

    pltpu.sync_copy(cid_hbm, cid_v)
    pltpu.sync_copy(rec_hbm, rec_v)
    pltpu.sync_copy(pts_hbm, pts_v)

    @pl.when(sidx == 0)
    def _():
        def zb(i, c):
            es_v[pl.ds(i * _L, _L)] = jnp.zeros((_L,), jnp.float32)
            return c

        lax.fori_loop(0, nv, zb, 0)

        def zs(i, c):
            pltpu.sync_copy(es_v, den_sh.at[pl.ds(i * _CH, _CH)])
            return c

        lax.fori_loop(0, n_points // _CH, zs, 0)

    plsc.subcore_barrier()

    def chunk_body(k, c):
        off = base + k * _CH
        pltpu.sync_copy(p0_hbm.at[pl.ds(off, _CH)], p0_v)
        pltpu.sync_copy(p1_hbm.at[pl.ds(off, _CH)], p1_v)

        def idx_body(i, cc):
            sl = pl.ds(i * _L, _L)
            p0c = p0_v[sl]
            p1c = p1_v[sl]
            lo = jnp.minimum(p0c, p1c)
            hi = jnp.maximum(p0c, p1c)
            bi = lo >> 9
            bj = hi >> 9
            rr = lo & 511
            cc_ = hi & 511
            tb = bi * n_blocks - ((bi * (bi - 1)) >> 1) + (bj - bi)
            qi_v[sl] = (tb * 262144
                        + ((cc_ >> 7) * 512 + rr) * 128 + (cc_ & 127))
            return cc

        lax.fori_loop(0, nv, idx_body, 0)

        pltpu.async_copy(q_hbm.at[qi_v], q_v, sem).wait()

        def cmp_body(i, cc):
            sl = pl.ds(i * _L, _L)
            p0c = p0_v[sl]
            p1c = p1_v[sl]
            c0 = plsc.load_gather(cid_v, [p0c])
            c1 = plsc.load_gather(cid_v, [p1c])
            r0 = plsc.load_gather(rec_v, [p0c])
            r1 = plsc.load_gather(rec_v, [p1c])
            t0 = plsc.load_gather(pts_v, [p0c])
            t1 = plsc.load_gather(pts_v, [p1c])
            pos = (c0 == c1) & (r0 > 0) & (r1 > 0) & (t0 > _PT) & (t1 > _PT)
            posf = jnp.where(pos, 1.0, 0.0)
            q = jnp.maximum(q_v[sl], 1e-20)
            ib = plsc.bitcast(q, jnp.int32)
            t = plsc.bitcast(0x5F3759DF - (ib >> 1), jnp.float32)
            t = t * (1.5 - 0.5 * q * t * t)
            t = t * (1.5 - 0.5 * q * t * t)
            t = t * (1.5 - 0.5 * q * t * t)
            d = q * t
            s = (1.0 / _TAU) * jnp.exp(d * (-1.0 / (2.0 * _SIGMA * _SIGMA)))
            es = jnp.exp(s - _SHIFT)
            es_v[sl] = es
            pos_v[sl] = posf
            nes_v[sl] = es * (1.0 - posf)
            return cc

        lax.fori_loop(0, nv, cmp_body, 0)

        pltpu.sync_copy(nes_v, den_sh.at[p0_v], add=True)
        pltpu.sync_copy(es_v, es_hbm.at[pl.ds(off, _CH)])
        pltpu.sync_copy(pos_v, pos_hbm.at[pl.ds(off, _CH)])
        return c

    lax.fori_loop(0, epw // _CH, chunk_body, 0)

    plsc.subcore_barrier()

    @pl.when(sidx == 0)
    def _():
        pltpu.sync_copy(den_sh, denp_hbm.at[cidx])


def _ln_poly(r):
    ib = plsc.bitcast(r, jnp.int32)
    e = ((ib >> 23) & 0xFF) - 127
    m = plsc.bitcast((ib & 0x7FFFFF) | (127 << 23), jnp.float32)
    big = m > 1.4142135623730951
    m = jnp.where(big, m * 0.5, m)
    ef = (e + jnp.where(big, 1, 0)).astype(jnp.float32)
    z = (m - 1.0) / (m + 1.0)
    z2 = z * z
    p = z * (2.0 + z2 * (0.66666667 + z2 * (0.4 + z2 * (0.28571429 + z2 * 0.22222222))))
    return ef * 0.6931471805599453 + p


def _sc_pass2_body(es_hbm, pos_hbm, p0_hbm, cid_hbm, denp_hbm,
                   labp_hbm,
                   den_v, dpb_v, cden_v, cid_v, p0_v, es_v, pos_v,
                   acc_v, cnt_v, row_v, *, n_points, n_pairs):
    cidx = lax.axis_index("c")
    sidx = lax.axis_index("s")
    wid = sidx * _NC + cidx
    epw = n_pairs // _NW
    base = wid * epw
    nv = _CH // _L
    nv_n = n_points // _L

    pltpu.sync_copy(cid_hbm, cid_v)
    pltpu.sync_copy(denp_hbm.at[0], den_v)
    pltpu.sync_copy(denp_hbm.at[1], dpb_v)

    def merge_body(i, c):
        sl = pl.ds(i * _L, _L)
        den_v[sl] = den_v[sl] + dpb_v[sl]
        return c

    lax.fori_loop(0, nv_n, merge_body, 0)

    def rank_body(i, carry):
        sl = pl.ds(i * _L, _L)
        dchunk = den_v[sl]
        pres = dchunk > 0.0
        presi = jnp.where(pres, 1, 0)
        cum = plsc.cumsum(presi)
        rank = carry + cum - 1
        plsc.store_scatter(cden_v, [rank], dchunk, mask=pres)
        return carry + jnp.max(cum)

    ng = lax.fori_loop(0, nv_n, rank_body, jnp.int32(0))
    ngm = jnp.maximum(ng, 1)

    def zb(i, c):
        sl = pl.ds(i * _L, _L)
        acc_v[sl] = jnp.zeros((_L,), jnp.float32)
        cnt_v[sl] = jnp.zeros((_L,), jnp.float32)
        return c

    lax.fori_loop(0, 1024 // _L, zb, 0)

    lane = lax.broadcasted_iota(jnp.int32, (_L,), 0)

    def chunk_body(k, c):
        off = base + k * _CH
        pltpu.sync_copy(p0_hbm.at[pl.ds(off, _CH)], p0_v)
        pltpu.sync_copy(es_hbm.at[pl.ds(off, _CH)], es_v)
        pltpu.sync_copy(pos_hbm.at[pl.ds(off, _CH)], pos_v)

        def body(i, cc):
            sl = pl.ds(i * _L, _L)
            p0c = p0_v[sl]
            num = es_v[sl]
            posf = pos_v[sl]
            di = jnp.minimum(p0c, ngm - 1)
            den = plsc.load_gather(cden_v, [di])
            r = num / (num + den)
            loss = -_ln_poly(r) * posf
            lab = jnp.clip(plsc.load_gather(cid_v, [p0c]), 0, 63)
            slot = lab * _L + lane
            plsc.addupdate_scatter(acc_v, [slot], loss)
            plsc.addupdate_scatter(cnt_v, [slot], posf)
            return cc

        lax.fori_loop(0, nv, body, 0)
        return c

    lax.fori_loop(0, epw // _CH, chunk_body, 0)

    def fold_body(g, c):
        base_lab = g * _L
        idx0 = (base_lab + lane) * _L

        def inner(k, vs):
            va, vc = vs
            va = va + plsc.load_gather(acc_v, [idx0 + k])
            vc = vc + plsc.load_gather(cnt_v, [idx0 + k])
            return (va, vc)

        va, vc = lax.fori_loop(
            0, _L, inner,
            (jnp.zeros((_L,), jnp.float32), jnp.zeros((_L,), jnp.float32)))
        row_v[pl.ds(base_lab, _L)] = va
        row_v[pl.ds(64 + base_lab, _L)] = vc
        return c

    lax.fori_loop(0, 64 // _L, fold_body, 0)

    pltpu.sync_copy(row_v, labp_hbm.at[wid])


def _fold_kernel(a_ref, o_ref):
    a = a_ref[...]
    srow = jnp.sum(a, axis=0, keepdims=True)
    ls = srow[:, :64]
    cs = srow[:, 64:]
    pres = cs > 0.0
    mean = jnp.where(pres, ls / jnp.where(pres, cs, 1.0), 0.0)
    total = jnp.sum(mean)
    nl = jnp.sum(jnp.where(pres, 1.0, 0.0))
    o_ref[0, 0] = total / nl


@jax.jit
def _impl(x, point_pairs, cluster_ids, recons, pts):
    n, d = x.shape
    e = point_pairs.shape[1]
    bm = bn = 512

    nb = pl.cdiv(n, bm)
    tri = [(bi, bj) for bi in range(nb) for bj in range(bi, nb)]
    nt = len(tri)
    bi_arr = jnp.array([t[0] for t in tri], dtype=jnp.int32)
    bj_arr = jnp.array([t[1] for t in tri], dtype=jnp.int32)
    q3 = pl.pallas_call(
        _dist_block_kernel,
        grid_spec=pltpu.PrefetchScalarGridSpec(
            num_scalar_prefetch=2,
            grid=(nt,),
            in_specs=[pl.BlockSpec((bm, d), lambda t, bi, bj: (bi[t], 0)),
                      pl.BlockSpec((bn, d), lambda t, bi, bj: (bj[t], 0))],
            out_specs=pl.BlockSpec((1, 4 * bm, 128),
                                   lambda t, bi, bj: (t, 0, 0)),
        ),
        out_shape=jax.ShapeDtypeStruct((nt, 4 * bm, 128), jnp.float32),
    )(bi_arr, bj_arr, x, x)
    qflat = q3.reshape(nt * 4 * bm * 128)

    p0 = point_pairs[0]
    p1 = point_pairs[1]
    cid = cluster_ids.astype(jnp.int32)
    rec = recons.astype(jnp.int32)
    pts_f = pts.astype(jnp.float32)

    mesh = plsc.VectorSubcoreMesh(core_axis_name="c", subcore_axis_name="s",
                                  num_cores=_NC, num_subcores=_NS)
    sc_params = pltpu.CompilerParams(needs_layout_passes=False)

    es, pos, denp = pl.kernel(
        functools.partial(_sc_pass1_body, n_points=n, n_pairs=e, n_blocks=nb),
        out_type=[jax.ShapeDtypeStruct((e,), jnp.float32),
                  jax.ShapeDtypeStruct((e,), jnp.float32),
                  jax.ShapeDtypeStruct((_NC, n), jnp.float32)],
        mesh=mesh,
        scratch_types=[
            pltpu.VMEM((n,), jnp.int32),
            pltpu.VMEM((n,), jnp.int32),
            pltpu.VMEM((n,), jnp.float32),
            pltpu.VMEM((_CH,), jnp.int32),
            pltpu.VMEM((_CH,), jnp.int32),
            pltpu.VMEM((_CH,), jnp.int32),
            pltpu.VMEM((_CH,), jnp.float32),
            pltpu.VMEM((_CH,), jnp.float32),
            pltpu.VMEM((_CH,), jnp.float32),
            pltpu.VMEM((_CH,), jnp.float32),
            pltpu.VMEM_SHARED((n,), jnp.float32),
            pltpu.SemaphoreType.DMA,
        ],
        compiler_params=sc_params,
    )(qflat, p0, p1, cid, rec, pts_f)

    labp = pl.kernel(
        functools.partial(_sc_pass2_body, n_points=n, n_pairs=e),
        out_type=[jax.ShapeDtypeStruct((_NW, 128), jnp.float32)],
        mesh=mesh,
        scratch_types=[
            pltpu.VMEM((n,), jnp.float32),
            pltpu.VMEM((n,), jnp.float32),
            pltpu.VMEM((n,), jnp.float32),
            pltpu.VMEM((n,), jnp.int32),
            pltpu.VMEM((_CH,), jnp.int32),
            pltpu.VMEM((_CH,), jnp.float32),
            pltpu.VMEM((_CH,), jnp.float32),
            pltpu.VMEM((1024,), jnp.float32),
            pltpu.VMEM((1024,), jnp.float32),
            pltpu.VMEM((128,), jnp.float32),
        ],
        compiler_params=sc_params,
    )(es, pos, p0, cid, denp)[0]

    out = pl.pallas_call(
        _fold_kernel,
        in_specs=[pl.BlockSpec((_NW, 128), lambda: (0, 0))],
        out_specs=pl.BlockSpec(memory_space=pltpu.SMEM),
        out_shape=jax.ShapeDtypeStruct((1, 1), jnp.float32),
    )(labp)
    return out[0, 0]


def kernel(x, point_pairs, cluster_ids, recons, pts):
    return _impl(x, point_pairs, cluster_ids, recons, pts)


# speedup vs baseline: 82.6253x; 1.3856x over previous
import functools

import jax
import jax.numpy as jnp
from jax import lax
from jax.experimental import pallas as pl
from jax.experimental.pallas import tpu as pltpu
from jax.experimental.pallas import tpu_sc as plsc

_TAU = 0.1
_SIGMA = 0.75
_PT = 0.9
_SHIFT = 10.0

_NC = 2
_NS = 16
_L = 16
_NW = _NC * _NS
_CH = 2000


def _dist_block_kernel(bi_ref, bj_ref, a_ref, b_ref, o_ref):
    a = a_ref[...]
    b = b_ref[...]
    ab = lax.dot_general(a, b, (((1,), (1,)), ((), ())),
                         preferred_element_type=jnp.float32)
    sqa = jnp.sum(a * a, axis=1, keepdims=True)
    sqb = jnp.sum(b * b, axis=1)[None, :]
    q = sqa + sqb - 2.0 * ab
    bm = a.shape[0]
    for t in range(bm // 128):
        o_ref[0, pl.ds(t * bm, bm), :] = q[:, t * 128:(t + 1) * 128]


def _sc_pass1_body(q_hbm, p0_hbm, p1_hbm, cid_hbm, rec_hbm, pts_hbm,
                   es_hbm, pos_hbm, denp_hbm,
                   cid_v, rec_v, pts_v, p0_v, p1_v, qi_v, q_v, es_v, pos_v,
                   nes_v, den_sh, sem, *, n_points, n_pairs, n_blocks,
                   bsz, bshift):
    cidx = lax.axis_index("c")
    sidx = lax.axis_index("s")
    wid = sidx * _NC + cidx
    epw = n_pairs // _NW
    base = wid * epw
    nv = _CH // _L

    pltpu.sync_copy(cid_hbm, cid_v)
    pltpu.sync_copy(rec_hbm, rec_v)
    pltpu.sync_copy(pts_hbm, pts_v)

    @pl.when(sidx == 0)
    def _():
        def zb(i, c):
            es_v[pl.ds(i * _L, _L)] = jnp.zeros((_L,), jnp.float32)
            return c

        lax.fori_loop(0, nv, zb, 0)

        def zs(i, c):
            pltpu.sync_copy(es_v, den_sh.at[pl.ds(i * _CH, _CH)])
            return c

        lax.fori_loop(0, n_points // _CH, zs, 0)

    plsc.subcore_barrier()

    def chunk_body(k, c):
        off = base + k * _CH
        pltpu.sync_copy(p0_hbm.at[pl.ds(off, _CH)], p0_v)
        pltpu.sync_copy(p1_hbm.at[pl.ds(off, _CH)], p1_v)

        def idx_body(i, cc):
            sl = pl.ds(i * _L, _L)
            p0c = p0_v[sl]
            p1c = p1_v[sl]
            lo = jnp.minimum(p0c, p1c)
            hi = jnp.maximum(p0c, p1c)
            bi = lo >> bshift
            bj = hi >> bshift
            rr = lo & (bsz - 1)
            cc_ = hi & (bsz - 1)
            tb = bi * n_blocks - ((bi * (bi - 1)) >> 1) + (bj - bi)
            qi_v[sl] = (tb * (bsz * bsz)
                        + ((cc_ >> 7) * bsz + rr) * 128 + (cc_ & 127))
            return cc

        lax.fori_loop(0, nv, idx_body, 0)

        pltpu.async_copy(q_hbm.at[qi_v], q_v, sem).wait()

        def cmp_body(i, cc):
            sl = pl.ds(i * _L, _L)
            p0c = p0_v[sl]
            p1c = p1_v[sl]
            c0 = plsc.load_gather(cid_v, [p0c])
            c1 = plsc.load_gather(cid_v, [p1c])
            r0 = plsc.load_gather(rec_v, [p0c])
            r1 = plsc.load_gather(rec_v, [p1c])
            t0 = plsc.load_gather(pts_v, [p0c])
            t1 = plsc.load_gather(pts_v, [p1c])
            pos = (c0 == c1) & (r0 > 0) & (r1 > 0) & (t0 > _PT) & (t1 > _PT)
            posf = jnp.where(pos, 1.0, 0.0)
            q = jnp.maximum(q_v[sl], 1e-20)
            ib = plsc.bitcast(q, jnp.int32)
            t = plsc.bitcast(0x5F3759DF - (ib >> 1), jnp.float32)
            t = t * (1.5 - 0.5 * q * t * t)
            t = t * (1.5 - 0.5 * q * t * t)
            t = t * (1.5 - 0.5 * q * t * t)
            d = q * t
            s = (1.0 / _TAU) * jnp.exp(d * (-1.0 / (2.0 * _SIGMA * _SIGMA)))
            es = jnp.exp(s - _SHIFT)
            es_v[sl] = es
            pos_v[sl] = posf
            nes_v[sl] = es * (1.0 - posf)
            return cc

        lax.fori_loop(0, nv, cmp_body, 0)

        pltpu.sync_copy(nes_v, den_sh.at[p0_v], add=True)
        pltpu.sync_copy(es_v, es_hbm.at[pl.ds(off, _CH)])
        pltpu.sync_copy(pos_v, pos_hbm.at[pl.ds(off, _CH)])
        return c

    lax.fori_loop(0, epw // _CH, chunk_body, 0)

    plsc.subcore_barrier()

    @pl.when(sidx == 0)
    def _():
        pltpu.sync_copy(den_sh, denp_hbm.at[cidx])


def _ln_poly(r):
    ib = plsc.bitcast(r, jnp.int32)
    e = ((ib >> 23) & 0xFF) - 127
    m = plsc.bitcast((ib & 0x7FFFFF) | (127 << 23), jnp.float32)
    big = m > 1.4142135623730951
    m = jnp.where(big, m * 0.5, m)
    ef = (e + jnp.where(big, 1, 0)).astype(jnp.float32)
    z = (m - 1.0) / (m + 1.0)
    z2 = z * z
    p = z * (2.0 + z2 * (0.66666667 + z2 * (0.4 + z2 * (0.28571429 + z2 * 0.22222222))))
    return ef * 0.6931471805599453 + p


def _sc_pass2_body(es_hbm, pos_hbm, p0_hbm, cid_hbm, denp_hbm,
                   labp_hbm,
                   den_v, dpb_v, cden_v, cid_v, p0_v, es_v, pos_v,
                   acc_v, cnt_v, row_v, *, n_points, n_pairs):
    cidx = lax.axis_index("c")
    sidx = lax.axis_index("s")
    wid = sidx * _NC + cidx
    epw = n_pairs // _NW
    base = wid * epw
    nv = _CH // _L
    nv_n = n_points // _L

    pltpu.sync_copy(cid_hbm, cid_v)
    pltpu.sync_copy(denp_hbm.at[0], den_v)
    pltpu.sync_copy(denp_hbm.at[1], dpb_v)

    def merge_body(i, c):
        sl = pl.ds(i * _L, _L)
        den_v[sl] = den_v[sl] + dpb_v[sl]
        return c

    lax.fori_loop(0, nv_n, merge_body, 0)

    def rank_body(i, carry):
        sl = pl.ds(i * _L, _L)
        dchunk = den_v[sl]
        pres = dchunk > 0.0
        presi = jnp.where(pres, 1, 0)
        cum = plsc.cumsum(presi)
        rank = carry + cum - 1
        plsc.store_scatter(cden_v, [rank], dchunk, mask=pres)
        return carry + jnp.max(cum)

    ng = lax.fori_loop(0, nv_n, rank_body, jnp.int32(0))
    ngm = jnp.maximum(ng, 1)

    def zb(i, c):
        sl = pl.ds(i * _L, _L)
        acc_v[sl] = jnp.zeros((_L,), jnp.float32)
        cnt_v[sl] = jnp.zeros((_L,), jnp.float32)
        return c

    lax.fori_loop(0, 1024 // _L, zb, 0)

    lane = lax.broadcasted_iota(jnp.int32, (_L,), 0)

    def chunk_body(k, c):
        off = base + k * _CH
        pltpu.sync_copy(p0_hbm.at[pl.ds(off, _CH)], p0_v)
        pltpu.sync_copy(es_hbm.at[pl.ds(off, _CH)], es_v)
        pltpu.sync_copy(pos_hbm.at[pl.ds(off, _CH)], pos_v)

        def body(i, cc):
            sl = pl.ds(i * _L, _L)
            p0c = p0_v[sl]
            num = es_v[sl]
            posf = pos_v[sl]
            di = jnp.minimum(p0c, ngm - 1)
            den = plsc.load_gather(cden_v, [di])
            r = num / (num + den)
            loss = -_ln_poly(r) * posf
            lab = jnp.clip(plsc.load_gather(cid_v, [p0c]), 0, 63)
            slot = lab * _L + lane
            plsc.addupdate_scatter(acc_v, [slot], loss)
            plsc.addupdate_scatter(cnt_v, [slot], posf)
            return cc

        lax.fori_loop(0, nv, body, 0)
        return c

    lax.fori_loop(0, epw // _CH, chunk_body, 0)

    def fold_body(g, c):
        base_lab = g * _L
        idx0 = (base_lab + lane) * _L

        def inner(k, vs):
            va, vc = vs
            va = va + plsc.load_gather(acc_v, [idx0 + k])
            vc = vc + plsc.load_gather(cnt_v, [idx0 + k])
            return (va, vc)

        va, vc = lax.fori_loop(
            0, _L, inner,
            (jnp.zeros((_L,), jnp.float32), jnp.zeros((_L,), jnp.float32)))
        row_v[pl.ds(base_lab, _L)] = va
        row_v[pl.ds(64 + base_lab, _L)] = vc
        return c

    lax.fori_loop(0, 64 // _L, fold_body, 0)

    pltpu.sync_copy(row_v, labp_hbm.at[wid])


def _fold_kernel(a_ref, o_ref):
    a = a_ref[...]
    srow = jnp.sum(a, axis=0, keepdims=True)
    ls = srow[:, :64]
    cs = srow[:, 64:]
    pres = cs > 0.0
    mean = jnp.where(pres, ls / jnp.where(pres, cs, 1.0), 0.0)
    total = jnp.sum(mean)
    nl = jnp.sum(jnp.where(pres, 1.0, 0.0))
    o_ref[0, 0] = total / nl


@jax.jit
def _impl(x, point_pairs, cluster_ids, recons, pts):
    n, d = x.shape
    e = point_pairs.shape[1]
    bm = bn = 1024
    bshift = 10

    nb = pl.cdiv(n, bm)
    tri = [(bi, bj) for bi in range(nb) for bj in range(bi, nb)]
    nt = len(tri)
    bi_arr = jnp.array([t[0] for t in tri], dtype=jnp.int32)
    bj_arr = jnp.array([t[1] for t in tri], dtype=jnp.int32)
    q3 = pl.pallas_call(
        _dist_block_kernel,
        grid_spec=pltpu.PrefetchScalarGridSpec(
            num_scalar_prefetch=2,
            grid=(nt,),
            in_specs=[pl.BlockSpec((bm, d), lambda t, bi, bj: (bi[t], 0)),
                      pl.BlockSpec((bn, d), lambda t, bi, bj: (bj[t], 0))],
            out_specs=pl.BlockSpec((1, bm * bm // 128, 128),
                                   lambda t, bi, bj: (t, 0, 0)),
        ),
        out_shape=jax.ShapeDtypeStruct((nt, bm * bm // 128, 128),
                                       jnp.float32),
    )(bi_arr, bj_arr, x, x)
    qflat = q3.reshape(nt * bm * bm)

    p0 = point_pairs[0]
    p1 = point_pairs[1]
    cid = cluster_ids.astype(jnp.int32)
    rec = recons.astype(jnp.int32)
    pts_f = pts.astype(jnp.float32)

    mesh = plsc.VectorSubcoreMesh(core_axis_name="c", subcore_axis_name="s",
                                  num_cores=_NC, num_subcores=_NS)
    sc_params = pltpu.CompilerParams(needs_layout_passes=False)

    es, pos, denp = pl.kernel(
        functools.partial(_sc_pass1_body, n_points=n, n_pairs=e, n_blocks=nb,
                          bsz=bm, bshift=bshift),
        out_type=[jax.ShapeDtypeStruct((e,), jnp.float32),
                  jax.ShapeDtypeStruct((e,), jnp.float32),
                  jax.ShapeDtypeStruct((_NC, n), jnp.float32)],
        mesh=mesh,
        scratch_types=[
            pltpu.VMEM((n,), jnp.int32),
            pltpu.VMEM((n,), jnp.int32),
            pltpu.VMEM((n,), jnp.float32),
            pltpu.VMEM((_CH,), jnp.int32),
            pltpu.VMEM((_CH,), jnp.int32),
            pltpu.VMEM((_CH,), jnp.int32),
            pltpu.VMEM((_CH,), jnp.float32),
            pltpu.VMEM((_CH,), jnp.float32),
            pltpu.VMEM((_CH,), jnp.float32),
            pltpu.VMEM((_CH,), jnp.float32),
            pltpu.VMEM_SHARED((n,), jnp.float32),
            pltpu.SemaphoreType.DMA,
        ],
        compiler_params=sc_params,
    )(qflat, p0, p1, cid, rec, pts_f)

    labp = pl.kernel(
        functools.partial(_sc_pass2_body, n_points=n, n_pairs=e),
        out_type=[jax.ShapeDtypeStruct((_NW, 128), jnp.float32)],
        mesh=mesh,
        scratch_types=[
            pltpu.VMEM((n,), jnp.float32),
            pltpu.VMEM((n,), jnp.float32),
            pltpu.VMEM((n,), jnp.float32),
            pltpu.VMEM((n,), jnp.int32),
            pltpu.VMEM((_CH,), jnp.int32),
            pltpu.VMEM((_CH,), jnp.float32),
            pltpu.VMEM((_CH,), jnp.float32),
            pltpu.VMEM((1024,), jnp.float32),
            pltpu.VMEM((1024,), jnp.float32),
            pltpu.VMEM((128,), jnp.float32),
        ],
        compiler_params=sc_params,
    )(es, pos, p0, cid, denp)[0]

    out = pl.pallas_call(
        _fold_kernel,
        in_specs=[pl.BlockSpec((_NW, 128), lambda: (0, 0))],
        out_specs=pl.BlockSpec(memory_space=pltpu.SMEM),
        out_shape=jax.ShapeDtypeStruct((1, 1), jnp.float32),
    )(labp)
    return out[0, 0]


def kernel(x, point_pairs, cluster_ids, recons, pts):
    return _impl(x, point_pairs, cluster_ids, recons, pts)
